# Initial kernel scaffold; baseline (speedup 1.0000x reference)
import jax
import jax.numpy as jnp
from jax.experimental import pallas as pl


def kernel(x, W_enc, W_dec, b_enc, b_dec, k):
    raise NotImplementedError("write your pallas kernel here")


# speedup vs baseline: 3.2144x; 3.2144x over previous
import functools

import jax
import jax.numpy as jnp
from jax import lax
from jax.experimental import pallas as pl
from jax.experimental.pallas import tpu as pltpu

D_IN, D_SAE, T, B = 768, 4096, 8, 8
NF = T * D_SAE
KMAX = 256
SB = 512


def _enc_body(x_ref, w_ref, b_ref, o_ref):
    for t in range(T):
        o_ref[:, t, :] = (
            jnp.dot(x_ref[:, t, :], w_ref[t], preferred_element_type=jnp.float32)
            + b_ref[0][None, :]
        )


def _encode(x, W_enc, b_enc2):
    return pl.pallas_call(
        _enc_body,
        grid=(D_SAE // SB,),
        in_specs=[
            pl.BlockSpec((B, T, D_IN), lambda s: (0, 0, 0)),
            pl.BlockSpec((T, D_IN, SB), lambda s: (0, 0, s)),
            pl.BlockSpec((1, SB), lambda s: (0, s)),
        ],
        out_specs=pl.BlockSpec((B, T, SB), lambda s: (0, 0, s)),
        out_shape=jax.ShapeDtypeStruct((B, T, D_SAE), jnp.float32),
    )(x, W_enc, b_enc2)


def _topk_body(k_ref, pre_ref, z_ref):
    pre = pre_ref[...]
    u = lax.bitcast_convert_type(pre, jnp.uint32)
    neg = (u >> 31) != 0
    key = jnp.where(neg, ~u, u | jnp.uint32(0x80000000))
    kk = jnp.minimum(k_ref[0], KMAX)

    def step(i, cur):
        bit = (jnp.uint32(1) << (jnp.uint32(31) - i.astype(jnp.uint32)))
        cand = cur | bit
        cnt = jnp.sum((key >= cand).astype(jnp.int32), axis=1, keepdims=True)
        return jnp.where(cnt >= kk, cand, cur)

    cur = lax.fori_loop(0, 32, step, jnp.zeros((B, 1), jnp.uint32))
    mask = key >= cur
    z_ref[...] = jnp.where(mask, jnp.maximum(pre, 0.0), 0.0)


def _topk_z(pre_flat, k_arr):
    return pl.pallas_call(
        _topk_body,
        in_specs=[
            pl.BlockSpec(memory_space=pltpu.SMEM),
            pl.BlockSpec(memory_space=pltpu.VMEM),
        ],
        out_specs=pl.BlockSpec(memory_space=pltpu.VMEM),
        out_shape=jax.ShapeDtypeStruct((B, NF), jnp.float32),
    )(k_arr, pre_flat)


def _dec_body(z_ref, w_ref, o_ref):
    s = pl.program_id(0)

    @pl.when(s == 0)
    def _():
        o_ref[...] = jnp.zeros_like(o_ref)

    for t in range(T):
        o_ref[:, t, :] += jnp.dot(
            z_ref[:, t, :], w_ref[:, t, :], preferred_element_type=jnp.float32
        )


def _decode_dense(z, W_dec):
    return pl.pallas_call(
        _dec_body,
        grid=(D_SAE // SB,),
        in_specs=[
            pl.BlockSpec((B, T, SB), lambda s: (0, 0, s)),
            pl.BlockSpec((SB, T, D_IN), lambda s: (s, 0, 0)),
        ],
        out_specs=pl.BlockSpec((B, T, D_IN), lambda s: (0, 0, 0)),
        out_shape=jax.ShapeDtypeStruct((B, T, D_IN), jnp.float32),
    )(z, W_dec)


def _fin_body(p_ref, b_ref, x_ref, xh_ref, loss_ref):
    xh = p_ref[...] + b_ref[...][None]
    xh_ref[...] = xh
    d = xh - x_ref[...]
    loss_ref[0, 0] = jnp.sum(d * d) / (B * T)


def _finalize(partial, b_dec, x):
    return pl.pallas_call(
        _fin_body,
        out_specs=(
            pl.BlockSpec(memory_space=pltpu.VMEM),
            pl.BlockSpec(memory_space=pltpu.SMEM),
        ),
        out_shape=(
            jax.ShapeDtypeStruct((B, T, D_IN), jnp.float32),
            jax.ShapeDtypeStruct((1, 1), jnp.float32),
        ),
    )(partial, b_dec, x)


def kernel(x, W_enc, W_dec, b_enc, b_dec, k):
    b_enc2 = b_enc.reshape(1, D_SAE)
    k_arr = jnp.asarray(k, jnp.int32).reshape(1)
    pre = _encode(x, W_enc, b_enc2)
    z_flat = _topk_z(pre.reshape(B, NF), k_arr)
    z = z_flat.reshape(B, T, D_SAE)
    partial = _decode_dense(z, W_dec)
    x_hat, loss = _finalize(partial, b_dec, x)
    return (loss.reshape(()), x_hat, z)
